# Initial kernel scaffold; baseline (speedup 1.0000x reference)
import jax
import jax.numpy as jnp
from jax.experimental import pallas as pl


def kernel(x, edge_index, edge_weight, edge_attr, mlp_w1, mlp_b1, mlp_w2, mlp_b2, lin1_w, lin2_w, lin2_b, lin_w, lin_b):
    raise NotImplementedError("write your pallas kernel here")


# speedup vs baseline: 2.3892x; 2.3892x over previous
import math

import jax
import jax.numpy as jnp
from jax import lax
from jax.experimental import pallas as pl
from jax.experimental.pallas import tpu as pltpu
from jax.experimental.pallas import tpu_sc as plsc

_N = 10000
_E = 320000
_H = 128
_F = 128
_G = 16
_CUTOFF = 10.0
_LOG2 = math.log(2.0)

_NC = 2
_NS = 16
_NW = _NC * _NS
_CH = 128
_NCHUNK = _E // _CH
_BASE = _NCHUNK // _NW
_EXTRA = _NCHUNK % _NW
_RPT = 624
_TAIL = _N - _NS * _RPT

_EB = 4000


def _ssp(v):
    return jnp.maximum(v, 0.0) + jnp.log(1.0 + jnp.exp(-jnp.abs(v))) - _LOG2


def _cutoff_body(ew_ref, c_ref):
    c_ref[...] = 0.5 * (jnp.cos(ew_ref[...] * (math.pi / _CUTOFF)) + 1.0)


def _edge_filter_body(ea_ref, c_ref, w1_ref, b1_ref, w2_ref, b2_ref, out_ref):
    t = jnp.dot(ea_ref[...], w1_ref[...], preferred_element_type=jnp.float32)
    t = _ssp(t + b1_ref[...])
    t = jnp.dot(t, w2_ref[...], preferred_element_type=jnp.float32) + b2_ref[...]
    out_ref[...] = t * c_ref[...]


def _embed_body(x_ref, w_ref, out_ref):
    out_ref[...] = jnp.dot(x_ref[...], w_ref[...], preferred_element_type=jnp.float32)


def _out_body(p_ref, w2_ref, b2_ref, lw_ref, lb_ref, o_ref):
    agg = p_ref[0] + p_ref[1]
    h2 = jnp.dot(agg, w2_ref[...], preferred_element_type=jnp.float32) + b2_ref[...]
    h2 = _ssp(h2)
    o_ref[...] = jnp.dot(h2, lw_ref[...], preferred_element_type=jnp.float32) + lb_ref[...]


def _sc_body(h_hbm, w_hbm, src_hbm, dst_hbm, out_hbm,
             agg_sh, rows_v, wv, sidx_v, didx_v, sem):
    c = lax.axis_index("c")
    s = lax.axis_index("s")
    w = c * _NS + s

    def _zero_row(i, _):
        for j in range(_H // 16):
            rows_v[i, pl.ds(16 * j, 16)] = jnp.zeros((16,), jnp.float32)
        return 0
    lax.fori_loop(0, _CH, _zero_row, 0)

    base = s * _RPT
    for k in range(_RPT // _CH):
        pltpu.sync_copy(rows_v, agg_sh.at[pl.ds(base + k * _CH, _CH)])
    rem = _RPT % _CH
    if rem:
        pltpu.sync_copy(rows_v.at[pl.ds(0, rem)],
                        agg_sh.at[pl.ds(base + (_RPT // _CH) * _CH, rem)])

    @pl.when(s == _NS - 1)
    def _zero_tail():
        pltpu.sync_copy(rows_v.at[pl.ds(0, _TAIL)],
                        agg_sh.at[pl.ds(_NS * _RPT, _TAIL)])
    plsc.subcore_barrier()

    start = w * _BASE + jnp.minimum(w, _EXTRA)
    count = jnp.where(w < _EXTRA, _BASE + 1, _BASE)

    def _chunk(g, _):
        @pl.when(g < count)
        def _do():
            row = start + g
            pltpu.sync_copy(src_hbm.at[row], sidx_v)
            pltpu.sync_copy(dst_hbm.at[row], didx_v)
            pltpu.async_copy(h_hbm.at[sidx_v], rows_v, sem).wait()
            pltpu.sync_copy(w_hbm.at[row], wv)

            def _mul(i, _2):
                for j in range(_H // 16):
                    sl = pl.ds(16 * j, 16)
                    rows_v[i, sl] = rows_v[i, sl] * wv[i, sl]
                return 0
            lax.fori_loop(0, _CH, _mul, 0)
            pltpu.sync_copy(rows_v, agg_sh.at[didx_v], add=True)
        return 0
    lax.fori_loop(0, _BASE + 1, _chunk, 0)

    plsc.subcore_barrier()
    pltpu.sync_copy(agg_sh.at[pl.ds(base, _RPT)],
                    out_hbm.at[c, pl.ds(base, _RPT)])

    @pl.when(s == _NS - 1)
    def _write_tail():
        pltpu.sync_copy(agg_sh.at[pl.ds(_NS * _RPT, _TAIL)],
                        out_hbm.at[c, pl.ds(_NS * _RPT, _TAIL)])


def kernel(x, edge_index, edge_weight, edge_attr,
           mlp_w1, mlp_b1, mlp_w2, mlp_b2,
           lin1_w, lin2_w, lin2_b, lin_w, lin_b):
    src = edge_index[0].astype(jnp.int32).reshape(_NCHUNK, _CH)
    dst = edge_index[1].astype(jnp.int32).reshape(_NCHUNK, _CH)

    cut = pl.pallas_call(
        _cutoff_body,
        out_shape=jax.ShapeDtypeStruct((_NCHUNK, _CH), jnp.float32),
    )(edge_weight.reshape(_NCHUNK, _CH))
    cut = cut.reshape(_E, 1)

    filt = pl.pallas_call(
        _edge_filter_body,
        grid=(_E // _EB,),
        in_specs=[
            pl.BlockSpec((_EB, _G), lambda i: (i, 0)),
            pl.BlockSpec((_EB, 1), lambda i: (i, 0)),
            pl.BlockSpec((_G, _F), lambda i: (0, 0)),
            pl.BlockSpec((1, _F), lambda i: (0, 0)),
            pl.BlockSpec((_F, _F), lambda i: (0, 0)),
            pl.BlockSpec((1, _F), lambda i: (0, 0)),
        ],
        out_specs=pl.BlockSpec((_EB, _F), lambda i: (i, 0)),
        out_shape=jax.ShapeDtypeStruct((_E, _F), jnp.float32),
    )(edge_attr, cut, mlp_w1, mlp_b1.reshape(1, _F), mlp_w2, mlp_b2.reshape(1, _F))

    h = pl.pallas_call(
        _embed_body,
        out_shape=jax.ShapeDtypeStruct((_N, _F), jnp.float32),
    )(x, lin1_w)

    sc = pl.kernel(
        _sc_body,
        out_type=jax.ShapeDtypeStruct((_NC, _N, _F), jnp.float32),
        mesh=plsc.VectorSubcoreMesh(core_axis_name="c", subcore_axis_name="s",
                                    num_cores=_NC, num_subcores=_NS),
        scratch_types=[
            pltpu.VMEM_SHARED((_N, _F), jnp.float32),
            pltpu.VMEM((_CH, _F), jnp.float32),
            pltpu.VMEM((_CH, _F), jnp.float32),
            pltpu.VMEM((_CH,), jnp.int32),
            pltpu.VMEM((_CH,), jnp.int32),
            pltpu.SemaphoreType.DMA,
        ],
    )
    partials = sc(h, filt.reshape(_NCHUNK, _CH, _F), src, dst)

    out = pl.pallas_call(
        _out_body,
        out_shape=jax.ShapeDtypeStruct((_N, _H), jnp.float32),
    )(partials, lin2_w, lin2_b.reshape(1, _H), lin_w, lin_b.reshape(1, _H))
    return out
